# Initial kernel scaffold; baseline (speedup 1.0000x reference)
import jax
import jax.numpy as jnp
from jax.experimental import pallas as pl


def kernel(x, Wg, W1, W2, W3, Ws1, Ws2, Ws3):
    raise NotImplementedError("write your pallas kernel here")


# speedup vs baseline: 1.5191x; 1.5191x over previous
import functools

import jax
import jax.numpy as jnp
from jax.experimental import pallas as pl
from jax.experimental.pallas import tpu as pltpu


def _router_body(x_ref, wg_ref, i0_ref, i1_ref, p0_ref, p1_ref):
    x = x_ref[...]
    wg = wg_ref[...]
    logits = jax.lax.dot_general(
        x, wg, (((1,), (1,)), ((), ())), preferred_element_type=jnp.float32
    )
    n, e = logits.shape
    eidx = jax.lax.broadcasted_iota(jnp.int32, (n, e), 1)
    m0 = jnp.max(logits, axis=1, keepdims=True)
    i0 = jnp.min(jnp.where(logits == m0, eidx, e), axis=1, keepdims=True)
    masked = jnp.where(eidx == i0, -jnp.inf, logits)
    m1 = jnp.max(masked, axis=1, keepdims=True)
    i1 = jnp.min(jnp.where(masked == m1, eidx, e), axis=1, keepdims=True)
    t = jnp.exp(m1 - m0)
    p0 = 1.0 / (1.0 + t)
    p1 = t * p0
    i0_ref[...] = i0
    i1_ref[...] = i1
    p0_ref[...] = p0
    p1_ref[...] = p1


def _moe_body(xb_ref, w1_ref, w2_ref, w3_ref, ws1_ref, ws2_ref, ws3_ref,
              i0_ref, i1_ref, p0_ref, p1_ref, y_ref):
    e = pl.program_id(0)
    x = xb_ref[...]

    def mlp(w1, w2, w3):
        h1 = jax.lax.dot(x, w1, preferred_element_type=jnp.float32)
        h2 = jax.lax.dot(x, w2, preferred_element_type=jnp.float32)
        h = h1 * (1.0 / (1.0 + jnp.exp(-h1))) * h2
        return jax.lax.dot(h.astype(jnp.bfloat16), w3,
                           preferred_element_type=jnp.float32)

    o = mlp(w1_ref[0].astype(jnp.bfloat16),
            w2_ref[0].astype(jnp.bfloat16),
            w3_ref[0].astype(jnp.bfloat16))
    wcol = (p0_ref[...] * (i0_ref[...] == e) +
            p1_ref[...] * (i1_ref[...] == e))

    @pl.when(e == 0)
    def _():
        s = mlp(ws1_ref[...].astype(jnp.bfloat16),
                ws2_ref[...].astype(jnp.bfloat16),
                ws3_ref[...].astype(jnp.bfloat16))
        y_ref[...] = s + o * wcol

    @pl.when(e != 0)
    def _():
        y_ref[...] += o * wcol


def kernel(x, Wg, W1, W2, W3, Ws1, Ws2, Ws3):
    Bb, Tt, Cc = x.shape
    E, _, I = W1.shape
    N = Bb * Tt
    x_flat = x.reshape(N, Cc)
    x_bf = x_flat.astype(jnp.bfloat16)

    i0, i1, p0, p1 = pl.pallas_call(
        _router_body,
        out_shape=(
            jax.ShapeDtypeStruct((N, 1), jnp.int32),
            jax.ShapeDtypeStruct((N, 1), jnp.int32),
            jax.ShapeDtypeStruct((N, 1), jnp.float32),
            jax.ShapeDtypeStruct((N, 1), jnp.float32),
        ),
    )(x_flat, Wg)

    grid = (E,)
    y = pl.pallas_call(
        _moe_body,
        grid=grid,
        in_specs=[
            pl.BlockSpec((N, Cc), lambda e: (0, 0)),
            pl.BlockSpec((1, Cc, I), lambda e: (e, 0, 0)),
            pl.BlockSpec((1, Cc, I), lambda e: (e, 0, 0)),
            pl.BlockSpec((1, I, Cc), lambda e: (e, 0, 0)),
            pl.BlockSpec((Cc, I), lambda e: (0, 0)),
            pl.BlockSpec((Cc, I), lambda e: (0, 0)),
            pl.BlockSpec((I, Cc), lambda e: (0, 0)),
            pl.BlockSpec((N, 1), lambda e: (0, 0)),
            pl.BlockSpec((N, 1), lambda e: (0, 0)),
            pl.BlockSpec((N, 1), lambda e: (0, 0)),
            pl.BlockSpec((N, 1), lambda e: (0, 0)),
        ],
        out_specs=pl.BlockSpec((N, Cc), lambda e: (0, 0)),
        out_shape=jax.ShapeDtypeStruct((N, Cc), jnp.float32),
    )(x_bf, W1, W2, W3, Ws1, Ws2, Ws3, i0, i1, p0, p1)

    return y.reshape(Bb, Tt, Cc)
